# Initial kernel scaffold; baseline (speedup 1.0000x reference)
import jax
import jax.numpy as jnp
from jax.experimental import pallas as pl


def kernel(x, sl):
    raise NotImplementedError("write your pallas kernel here")


# speedup vs baseline: 152.1774x; 152.1774x over previous
import functools

import jax
import jax.numpy as jnp
from jax import lax
from jax.experimental import pallas as pl
from jax.experimental.pallas import tpu as pltpu
from jax.experimental.pallas import tpu_sc as plsc

N = 320000
D = 128
C = 10000
B = 64
NBLK = N // B
NC = 2
NS = 16
NW = NC * NS
FULL = NBLK // NW
EXTRA = NBLK % NW
TILE_SHARE = 624
TAIL_ROWS = C - NS * TILE_SHARE


def _count_chunks(ib, cnt):
  for k in range(B // 16):
    ids = ib[pl.ds(k * 16, 16)]
    run, last = plsc.scan_count(ids)
    plsc.addupdate_scatter(cnt, [ids], run.astype(jnp.float32), mask=last)


def _sc_body(x_hbm, sl_hbm, z128_hbm, psums, pcnts,
             sums_s, xb0, xb1, ib0, ib1, zb, cnt,
             sx0, sx1, si0, si1):
  cid = lax.axis_index("c")
  sid = lax.axis_index("s")
  wid = sid * NC + cid
  nvalid = jnp.where(wid < EXTRA, FULL + 1, FULL)

  r0 = pl.multiple_of(wid * B, B)
  r1 = pl.multiple_of((wid + NW) * B, B)
  pltpu.async_copy(x_hbm.at[pl.ds(r0, B)], xb0, sx0)
  pltpu.async_copy(sl_hbm.at[pl.ds(r0, B)], ib0, si0)
  pltpu.async_copy(x_hbm.at[pl.ds(r1, B)], xb1, sx1)
  pltpu.async_copy(sl_hbm.at[pl.ds(r1, B)], ib1, si1)

  pltpu.sync_copy(z128_hbm, zb)
  base = pl.multiple_of(sid * TILE_SHARE, 8)
  for k in range(39):
    off = pl.multiple_of(base + k * 16, 8)
    pltpu.sync_copy(zb, sums_s.at[pl.ds(off, 16)])

  @pl.when(sid == NS - 1)
  def _():
    pltpu.sync_copy(zb, sums_s.at[pl.ds(NS * TILE_SHARE, TAIL_ROWS)])

  def zero_cnt(i, _):
    cnt[pl.ds(i * 16, 16)] = jnp.zeros((16,), jnp.float32)
    return 0
  lax.fori_loop(0, C // 16, zero_cnt, 0)

  plsc.subcore_barrier()

  def step(t, _):
    for par, (xb, ib, sx, si) in enumerate(
        ((xb0, ib0, sx0, si0), (xb1, ib1, sx1, si1))):
      j = 2 * t + par
      pltpu.make_async_copy(x_hbm.at[pl.ds(0, B)], xb, sx).wait()
      pltpu.make_async_copy(sl_hbm.at[pl.ds(0, B)], ib, si).wait()

      pltpu.sync_copy(xb, sums_s.at[ib], add=True)
      _count_chunks(ib, cnt)

      nxt = j + 2

      @pl.when(nxt < nvalid)
      def _():
        blk = pl.multiple_of((nxt * NW + wid) * B, B)
        pltpu.async_copy(x_hbm.at[pl.ds(blk, B)], xb, sx)
        pltpu.async_copy(sl_hbm.at[pl.ds(blk, B)], ib, si)
    return 0
  lax.fori_loop(0, FULL // 2, step, 0)

  @pl.when(wid < EXTRA)
  def _():
    pltpu.make_async_copy(x_hbm.at[pl.ds(0, B)], xb0, sx0).wait()
    pltpu.make_async_copy(sl_hbm.at[pl.ds(0, B)], ib0, si0).wait()
    pltpu.sync_copy(xb0, sums_s.at[ib0], add=True)
    _count_chunks(ib0, cnt)

  plsc.subcore_barrier()

  pltpu.sync_copy(sums_s.at[pl.ds(base, TILE_SHARE)],
                  psums.at[cid, pl.ds(base, TILE_SHARE)])

  @pl.when(sid == NS - 1)
  def _():
    pltpu.sync_copy(sums_s.at[pl.ds(NS * TILE_SHARE, TAIL_ROWS)],
                    psums.at[cid, pl.ds(NS * TILE_SHARE, TAIL_ROWS)])

  cof = pl.multiple_of(wid * C, 8)
  pltpu.sync_copy(cnt, pcnts.at[pl.ds(cof, C)])


_sc_stage = functools.partial(
    pl.kernel,
    out_type=(jax.ShapeDtypeStruct((NC, C, D), jnp.float32),
              jax.ShapeDtypeStruct((NW * C,), jnp.float32)),
    mesh=plsc.VectorSubcoreMesh(core_axis_name="c", subcore_axis_name="s"),
    compiler_params=pltpu.CompilerParams(needs_layout_passes=False),
    scratch_types=(
        pltpu.VMEM_SHARED((C, D), jnp.float32),
        pltpu.VMEM((B, D), jnp.float32),
        pltpu.VMEM((B, D), jnp.float32),
        pltpu.VMEM((B,), jnp.int32),
        pltpu.VMEM((B,), jnp.int32),
        pltpu.VMEM((16, D), jnp.float32),
        pltpu.VMEM((C,), jnp.float32),
        pltpu.SemaphoreType.DMA,
        pltpu.SemaphoreType.DMA,
        pltpu.SemaphoreType.DMA,
        pltpu.SemaphoreType.DMA,
    ),
)(_sc_body)


def _div_body(ps_ref, pc_ref, out_ref):
  s = ps_ref[0] + ps_ref[1]
  n = jnp.sum(pc_ref[...], axis=0)
  out_ref[...] = s / n[:, None]


def kernel(x, sl):
  sl32 = sl.astype(jnp.int32)
  z128 = jnp.zeros((16, D), jnp.float32)
  psums, pcnts = _sc_stage(x, sl32, z128)
  out = pl.pallas_call(
      _div_body,
      out_shape=jax.ShapeDtypeStruct((C, D), jnp.float32),
  )(psums, pcnts.reshape(NW, C))
  return out
